# baseline (speedup 1.0000x reference)
import functools

import jax
import jax.numpy as jnp
from jax import lax
from jax.experimental import pallas as pl
from jax.experimental.pallas import tpu as pltpu
from jax.experimental.pallas import tpu_sc as plsc

N = 10000
E = 320000
D = 128
NP = 10240
NCORE = 2
NSUB = 16
TILES = NCORE * NSUB
PER_TILE = 10240
EP = TILES * PER_TILE
PAD = EP - E
ER = EP // 128

@functools.cache
def _get_mesh():
    return plsc.VectorSubcoreMesh(
        core_axis_name="c", subcore_axis_name="s",
        num_cores=NCORE, num_subcores=NSUB)


@functools.cache
def _get_deg_kernel():
    return functools.partial(
        pl.kernel,
        out_type=jax.ShapeDtypeStruct((NCORE, 2, NP), jnp.float32),
        mesh=_get_mesh(),
        scratch_types=[
            pltpu.VMEM((8, 128), jnp.int32),
            pltpu.VMEM((8, 128), jnp.int32),
            pltpu.VMEM((1024,), jnp.float32),
            pltpu.VMEM_SHARED((NP,), jnp.float32),
            pltpu.VMEM_SHARED((NP,), jnp.float32),
        ],
    )(_deg_body)


def _deg_body(src2_h, dst2_h, ones_h, zero1_h, out_h,
              iv_s, iv_d, ones_v, accs, accd):
    cid = lax.axis_index("c")
    sid = lax.axis_index("s")
    tid = cid * NSUB + sid
    pltpu.sync_copy(ones_h, ones_v)

    @pl.when(sid == 0)
    def _():
        pltpu.sync_copy(zero1_h, accs)

    @pl.when(sid == 1)
    def _():
        pltpu.sync_copy(zero1_h, accd)

    plsc.subcore_barrier()

    def body(i, carry):
        r0 = tid * (PER_TILE // 128) + i * 8
        pltpu.sync_copy(src2_h.at[pl.ds(r0, 8), :], iv_s)
        pltpu.sync_copy(dst2_h.at[pl.ds(r0, 8), :], iv_d)
        for j in range(8):
            pltpu.sync_copy(ones_v.at[pl.ds(j * 128, 128)],
                            accs.at[iv_s.at[j]], add=True)
            pltpu.sync_copy(ones_v.at[pl.ds(j * 128, 128)],
                            accd.at[iv_d.at[j]], add=True)
        return carry

    lax.fori_loop(0, PER_TILE // 1024, body, 0)
    plsc.subcore_barrier()
    pltpu.sync_copy(accs.at[pl.ds(sid * (NP // NSUB), NP // NSUB)],
                    out_h.at[cid, 0, pl.ds(sid * (NP // NSUB), NP // NSUB)])
    pltpu.sync_copy(accd.at[pl.ds(sid * (NP // NSUB), NP // NSUB)],
                    out_h.at[cid, 1, pl.ds(sid * (NP // NSUB), NP // NSUB)])


_CH = 128
_NCH = PER_TILE // _CH


@functools.cache
def _get_agg_kernel():
    return functools.partial(
        pl.kernel,
        out_type=jax.ShapeDtypeStruct((NCORE, NP, D), jnp.float32),
        mesh=_get_mesh(),
        scratch_types=[
            pltpu.VMEM((_CH,), jnp.int32),
            pltpu.VMEM((_CH,), jnp.int32),
            pltpu.VMEM((1, 128), jnp.int32),
            pltpu.VMEM((1, 128), jnp.int32),
            pltpu.VMEM((_CH, D), jnp.float32),
            pltpu.VMEM((_CH, D), jnp.float32),
            pltpu.VMEM_SHARED((NP, D), jnp.float32),
            pltpu.SemaphoreType.DMA,
            pltpu.SemaphoreType.DMA,
            pltpu.SemaphoreType.DMA,
            pltpu.SemaphoreType.DMA,
        ],
    )(_agg_body)


def _agg_body(y_h, src_h, dst2_h, zrow_h, out_h,
              sv0, sv1, dv0, dv1, rows0, rows1, acc,
              gsem0, gsem1, ssem0, ssem1):
    cid = lax.axis_index("c")
    sid = lax.axis_index("s")
    tid = cid * NSUB + sid
    stripe = NP // NSUB
    pltpu.sync_copy(zrow_h, acc.at[pl.ds(sid * stripe, stripe), :])
    plsc.subcore_barrier()
    ebase = tid * PER_TILE
    rbase = tid * (PER_TILE // 128)

    def load_idx(i, sv, dv):
        pltpu.sync_copy(src_h.at[pl.ds(ebase + i * _CH, _CH)], sv)
        pltpu.sync_copy(dst2_h.at[pl.ds(rbase + i, 1), :], dv)

    load_idx(0, sv0, dv0)
    pltpu.async_copy(y_h.at[sv0], rows0, gsem0)
    nsteps = _NCH // 2

    def step(s, carry):
        i0 = s * 2
        pltpu.make_async_copy(y_h.at[sv0], rows0, gsem0).wait()

        @pl.when(s > 0)
        def _():
            pltpu.make_async_copy(rows1, acc.at[dv1.at[0]], ssem1).wait()

        load_idx(i0 + 1, sv1, dv1)
        pltpu.async_copy(y_h.at[sv1], rows1, gsem1)
        pltpu.async_copy(rows0, acc.at[dv0.at[0]], ssem0, add=True)
        pltpu.make_async_copy(y_h.at[sv1], rows1, gsem1).wait()
        pltpu.make_async_copy(rows0, acc.at[dv0.at[0]], ssem0).wait()

        @pl.when(s + 1 < nsteps)
        def _():
            load_idx(i0 + 2, sv0, dv0)
            pltpu.async_copy(y_h.at[sv0], rows0, gsem0)

        pltpu.async_copy(rows1, acc.at[dv1.at[0]], ssem1, add=True)
        return carry

    lax.fori_loop(0, nsteps, step, 0)
    pltpu.make_async_copy(rows1, acc.at[dv1.at[0]], ssem1).wait()
    plsc.subcore_barrier()
    pltpu.sync_copy(acc.at[pl.ds(sid * stripe, stripe), :],
                    out_h.at[cid, pl.ds(sid * stripe, stripe), :])


def _norms(degs, which):
    d = degs[0, which] + degs[1, which]
    return jnp.where(d > 0, lax.rsqrt(d), 0.0)


def _tc_a_body(hp_ref, degs_ref, w1_ref, y_ref, sumh_ref):
    degs = degs_ref[...]
    ns = _norms(degs, 0)
    hp = hp_ref[...]
    y_ref[...] = jnp.dot(hp * ns[:, None], w1_ref[...],
                         preferred_element_type=jnp.float32)
    sumh_ref[...] = jnp.sum(hp, axis=0, keepdims=True)


def _bn_relu_masked(parts, nd, g, be):
    a = (parts[0] + parts[1]) * nd[:, None]
    m = jnp.sum(a, axis=0, keepdims=True) / N
    msq = jnp.sum(a * a, axis=0, keepdims=True) / N
    v = msq - m * m
    z = (a - m) * lax.rsqrt(v + 1e-5) * g + be
    z = jnp.maximum(z, 0.0)
    mask = lax.broadcasted_iota(jnp.int32, (NP, 1), 0) < N
    return jnp.where(mask, z, 0.0)


def _tc_b_body(parts_ref, degs_ref, w2_ref, g1_ref, be1_ref, y2_ref, sumz_ref):
    degs = degs_ref[...]
    nd = _norms(degs, 1)
    ns = _norms(degs, 0)
    z = _bn_relu_masked(parts_ref[...], nd, g1_ref[...], be1_ref[...])
    sumz_ref[...] = jnp.sum(z, axis=0, keepdims=True)
    y2_ref[...] = jnp.dot(z * ns[:, None], w2_ref[...],
                          preferred_element_type=jnp.float32)


def _tc_c_body(parts_ref, degs_ref, g2_ref, be2_ref, sumh_ref, sumz1_ref,
               pw0_ref, pb0_ref, pw1_ref, pb1_ref, pw2_ref, pb2_ref, out_ref):
    degs = degs_ref[...]
    nd = _norms(degs, 1)
    z2 = _bn_relu_masked(parts_ref[...], nd, g2_ref[...], be2_ref[...])
    s2 = jnp.sum(z2, axis=0, keepdims=True)
    out_ref[...] = (
        jnp.dot(sumh_ref[...], pw0_ref[...], preferred_element_type=jnp.float32)
        + pb0_ref[...]
        + jnp.dot(sumz1_ref[...], pw1_ref[...], preferred_element_type=jnp.float32)
        + pb1_ref[...]
        + jnp.dot(s2, pw2_ref[...], preferred_element_type=jnp.float32)
        + pb2_ref[...])


_tc_a = pl.pallas_call(
    _tc_a_body,
    out_shape=(jax.ShapeDtypeStruct((NP, D), jnp.float32),
               jax.ShapeDtypeStruct((1, D), jnp.float32)))

_tc_b = pl.pallas_call(
    _tc_b_body,
    out_shape=(jax.ShapeDtypeStruct((NP, D), jnp.float32),
               jax.ShapeDtypeStruct((1, D), jnp.float32)))

_tc_c = pl.pallas_call(
    _tc_c_body,
    out_shape=jax.ShapeDtypeStruct((1, D), jnp.float32))


def kernel(h, edge_index, W1, b1, W2, b2, g1, be1, g2, be2,
           PW0, PB0, PW1, PB1, PW2, PB2):
    pad_vals = (N + (jnp.arange(PAD, dtype=jnp.int32) % (NP - N))).astype(jnp.int32)
    srcp = jnp.concatenate([edge_index[0], pad_vals])
    dstp = jnp.concatenate([edge_index[1], pad_vals])
    src2 = srcp.reshape(ER, 128)
    dst2 = dstp.reshape(ER, 128)
    hp = jnp.pad(h, ((0, NP - N), (0, 0)))
    zrow = jnp.zeros((NP // NSUB, D), jnp.float32)
    zero1 = jnp.zeros((NP,), jnp.float32)
    ones1k = jnp.ones((1024,), jnp.float32)

    degs = _get_deg_kernel()(src2, dst2, ones1k, zero1)
    y1, sumh = _tc_a(hp, degs, W1)
    parts1 = _get_agg_kernel()(y1, srcp, dst2, zrow)
    y2, sumz1 = _tc_b(parts1, degs, W2, g1, be1)
    parts2 = _get_agg_kernel()(y2, srcp, dst2, zrow)
    return _tc_c(parts2, degs, g2, be2, sumh, sumz1,
                 PW0, PB0, PW1, PB1, PW2, PB2)


# speedup vs baseline: 1.0352x; 1.0352x over previous
import functools

import jax
import jax.numpy as jnp
from jax import lax
from jax.experimental import pallas as pl
from jax.experimental.pallas import tpu as pltpu
from jax.experimental.pallas import tpu_sc as plsc

N = 10000
E = 320000
D = 128
NP = 10240
NCORE = 2
NSUB = 16
TILES = NCORE * NSUB
PER_TILE = 10240
EP = TILES * PER_TILE
PAD = EP - E
ER = EP // 128

@functools.cache
def _get_mesh():
    return plsc.VectorSubcoreMesh(
        core_axis_name="c", subcore_axis_name="s",
        num_cores=NCORE, num_subcores=NSUB)


@functools.cache
def _get_deg_kernel():
    return functools.partial(
        pl.kernel,
        out_type=jax.ShapeDtypeStruct((NCORE, 2, NP), jnp.float32),
        mesh=_get_mesh(),
        scratch_types=[
            pltpu.VMEM((8, 128), jnp.int32),
            pltpu.VMEM((8, 128), jnp.int32),
            pltpu.VMEM((1024,), jnp.float32),
            pltpu.VMEM_SHARED((NP,), jnp.float32),
            pltpu.VMEM_SHARED((NP,), jnp.float32),
        ],
    )(_deg_body)


def _deg_body(src2_h, dst2_h, ones_h, zero1_h, out_h,
              iv_s, iv_d, ones_v, accs, accd):
    cid = lax.axis_index("c")
    sid = lax.axis_index("s")
    tid = cid * NSUB + sid
    pltpu.sync_copy(ones_h, ones_v)

    @pl.when(sid == 0)
    def _():
        pltpu.sync_copy(zero1_h, accs)

    @pl.when(sid == 1)
    def _():
        pltpu.sync_copy(zero1_h, accd)

    plsc.subcore_barrier()

    def body(i, carry):
        r0 = tid * (PER_TILE // 128) + i * 8
        pltpu.sync_copy(src2_h.at[pl.ds(r0, 8), :], iv_s)
        pltpu.sync_copy(dst2_h.at[pl.ds(r0, 8), :], iv_d)
        for j in range(8):
            pltpu.sync_copy(ones_v.at[pl.ds(j * 128, 128)],
                            accs.at[iv_s.at[j]], add=True)
            pltpu.sync_copy(ones_v.at[pl.ds(j * 128, 128)],
                            accd.at[iv_d.at[j]], add=True)
        return carry

    lax.fori_loop(0, PER_TILE // 1024, body, 0)
    plsc.subcore_barrier()
    pltpu.sync_copy(accs.at[pl.ds(sid * (NP // NSUB), NP // NSUB)],
                    out_h.at[cid, 0, pl.ds(sid * (NP // NSUB), NP // NSUB)])
    pltpu.sync_copy(accd.at[pl.ds(sid * (NP // NSUB), NP // NSUB)],
                    out_h.at[cid, 1, pl.ds(sid * (NP // NSUB), NP // NSUB)])


_CH = 256
_NCH = PER_TILE // _CH


@functools.cache
def _get_agg_kernel():
    return functools.partial(
        pl.kernel,
        out_type=jax.ShapeDtypeStruct((NCORE, NP, D), jnp.float32),
        mesh=_get_mesh(),
        scratch_types=[
            pltpu.VMEM((_CH,), jnp.int32),
            pltpu.VMEM((_CH // 128, 128), jnp.int32),
            pltpu.VMEM((_CH, D), jnp.float32),
            pltpu.VMEM_SHARED((NP, D), jnp.float32),
            pltpu.SemaphoreType.DMA,
            pltpu.SemaphoreType.DMA,
        ],
    )(_agg_body)


def _agg_body(y_h, src_h, dst2_h, zrow_h, out_h, sv, dv, rows, acc, sem, sem2):
    cid = lax.axis_index("c")
    sid = lax.axis_index("s")
    tid = cid * NSUB + sid
    stripe = NP // NSUB
    pltpu.sync_copy(zrow_h, acc.at[pl.ds(sid * stripe, stripe), :])
    plsc.subcore_barrier()

    def body(i, carry):
        off = tid * PER_TILE + i * _CH
        pltpu.sync_copy(src_h.at[pl.ds(off, _CH)], sv)
        r0 = tid * (PER_TILE // 128) + i * (_CH // 128)
        pltpu.sync_copy(dst2_h.at[pl.ds(r0, _CH // 128), :], dv)
        pltpu.async_copy(y_h.at[sv.at[pl.ds(0, _CH // 2)]],
                         rows.at[pl.ds(0, _CH // 2), :], sem)
        pltpu.async_copy(y_h.at[sv.at[pl.ds(_CH // 2, _CH // 2)]],
                         rows.at[pl.ds(_CH // 2, _CH // 2), :], sem2)
        pltpu.make_async_copy(y_h.at[sv.at[pl.ds(0, _CH // 2)]],
                              rows.at[pl.ds(0, _CH // 2), :], sem).wait()
        pltpu.make_async_copy(y_h.at[sv.at[pl.ds(_CH // 2, _CH // 2)]],
                              rows.at[pl.ds(_CH // 2, _CH // 2), :], sem2).wait()
        for j in range(_CH // 128):
            pltpu.sync_copy(rows.at[pl.ds(j * 128, 128), :],
                            acc.at[dv.at[j]], add=True)
        return carry

    lax.fori_loop(0, _NCH, body, 0)
    plsc.subcore_barrier()
    pltpu.sync_copy(acc.at[pl.ds(sid * stripe, stripe), :],
                    out_h.at[cid, pl.ds(sid * stripe, stripe), :])


def _norms(degs, which):
    d = degs[0, which] + degs[1, which]
    return jnp.where(d > 0, lax.rsqrt(d), 0.0)


def _tc_a_body(hp_ref, degs_ref, w1_ref, y_ref, sumh_ref):
    degs = degs_ref[...]
    ns = _norms(degs, 0)
    hp = hp_ref[...]
    y_ref[...] = jnp.dot(hp * ns[:, None], w1_ref[...],
                         preferred_element_type=jnp.float32)
    sumh_ref[...] = jnp.sum(hp, axis=0, keepdims=True)


def _bn_relu_masked(parts, nd, g, be):
    a = (parts[0] + parts[1]) * nd[:, None]
    m = jnp.sum(a, axis=0, keepdims=True) / N
    msq = jnp.sum(a * a, axis=0, keepdims=True) / N
    v = msq - m * m
    z = (a - m) * lax.rsqrt(v + 1e-5) * g + be
    z = jnp.maximum(z, 0.0)
    mask = lax.broadcasted_iota(jnp.int32, (NP, 1), 0) < N
    return jnp.where(mask, z, 0.0)


def _tc_b_body(parts_ref, degs_ref, w2_ref, g1_ref, be1_ref, y2_ref, sumz_ref):
    degs = degs_ref[...]
    nd = _norms(degs, 1)
    ns = _norms(degs, 0)
    z = _bn_relu_masked(parts_ref[...], nd, g1_ref[...], be1_ref[...])
    sumz_ref[...] = jnp.sum(z, axis=0, keepdims=True)
    y2_ref[...] = jnp.dot(z * ns[:, None], w2_ref[...],
                          preferred_element_type=jnp.float32)


def _tc_c_body(parts_ref, degs_ref, g2_ref, be2_ref, sumh_ref, sumz1_ref,
               pw0_ref, pb0_ref, pw1_ref, pb1_ref, pw2_ref, pb2_ref, out_ref):
    degs = degs_ref[...]
    nd = _norms(degs, 1)
    z2 = _bn_relu_masked(parts_ref[...], nd, g2_ref[...], be2_ref[...])
    s2 = jnp.sum(z2, axis=0, keepdims=True)
    out_ref[...] = (
        jnp.dot(sumh_ref[...], pw0_ref[...], preferred_element_type=jnp.float32)
        + pb0_ref[...]
        + jnp.dot(sumz1_ref[...], pw1_ref[...], preferred_element_type=jnp.float32)
        + pb1_ref[...]
        + jnp.dot(s2, pw2_ref[...], preferred_element_type=jnp.float32)
        + pb2_ref[...])


_tc_a = pl.pallas_call(
    _tc_a_body,
    out_shape=(jax.ShapeDtypeStruct((NP, D), jnp.float32),
               jax.ShapeDtypeStruct((1, D), jnp.float32)))

_tc_b = pl.pallas_call(
    _tc_b_body,
    out_shape=(jax.ShapeDtypeStruct((NP, D), jnp.float32),
               jax.ShapeDtypeStruct((1, D), jnp.float32)))

_tc_c = pl.pallas_call(
    _tc_c_body,
    out_shape=jax.ShapeDtypeStruct((1, D), jnp.float32))


def kernel(h, edge_index, W1, b1, W2, b2, g1, be1, g2, be2,
           PW0, PB0, PW1, PB1, PW2, PB2):
    pad_vals = (N + (jnp.arange(PAD, dtype=jnp.int32) % (NP - N))).astype(jnp.int32)
    srcp = jnp.concatenate([edge_index[0], pad_vals])
    dstp = jnp.concatenate([edge_index[1], pad_vals])
    src2 = srcp.reshape(ER, 128)
    dst2 = dstp.reshape(ER, 128)
    hp = jnp.pad(h, ((0, NP - N), (0, 0)))
    zrow = jnp.zeros((NP // NSUB, D), jnp.float32)
    zero1 = jnp.zeros((NP,), jnp.float32)
    ones1k = jnp.ones((1024,), jnp.float32)

    degs = _get_deg_kernel()(src2, dst2, ones1k, zero1)
    y1, sumh = _tc_a(hp, degs, W1)
    parts1 = _get_agg_kernel()(y1, srcp, dst2, zrow)
    y2, sumz1 = _tc_b(parts1, degs, W2, g1, be1)
    parts2 = _get_agg_kernel()(y2, srcp, dst2, zrow)
    return _tc_c(parts2, degs, g2, be2, sumh, sumz1,
                 PW0, PB0, PW1, PB1, PW2, PB2)


# speedup vs baseline: 1.0428x; 1.0074x over previous
import functools

import jax
import jax.numpy as jnp
from jax import lax
from jax.experimental import pallas as pl
from jax.experimental.pallas import tpu as pltpu
from jax.experimental.pallas import tpu_sc as plsc

N = 10000
E = 320000
D = 128
NP = 10240
NCORE = 2
NSUB = 16
TILES = NCORE * NSUB
PER_TILE = 10240
EP = TILES * PER_TILE
PAD = EP - E
ER = EP // 128

@functools.cache
def _get_mesh():
    return plsc.VectorSubcoreMesh(
        core_axis_name="c", subcore_axis_name="s",
        num_cores=NCORE, num_subcores=NSUB)


@functools.cache
def _get_deg_kernel():
    return functools.partial(
        pl.kernel,
        out_type=jax.ShapeDtypeStruct((NCORE, 2, NP), jnp.float32),
        mesh=_get_mesh(),
        scratch_types=[
            pltpu.VMEM((8, 128), jnp.int32),
            pltpu.VMEM((8, 128), jnp.int32),
            pltpu.VMEM((1024,), jnp.float32),
            pltpu.VMEM_SHARED((NP,), jnp.float32),
            pltpu.VMEM_SHARED((NP,), jnp.float32),
        ],
    )(_deg_body)


def _deg_body(src2_h, dst2_h, ones_h, zero1_h, out_h,
              iv_s, iv_d, ones_v, accs, accd):
    cid = lax.axis_index("c")
    sid = lax.axis_index("s")
    tid = cid * NSUB + sid
    pltpu.sync_copy(ones_h, ones_v)

    @pl.when(sid == 0)
    def _():
        pltpu.sync_copy(zero1_h, accs)

    @pl.when(sid == 1)
    def _():
        pltpu.sync_copy(zero1_h, accd)

    plsc.subcore_barrier()

    def body(i, carry):
        r0 = tid * (PER_TILE // 128) + i * 8
        pltpu.sync_copy(src2_h.at[pl.ds(r0, 8), :], iv_s)
        pltpu.sync_copy(dst2_h.at[pl.ds(r0, 8), :], iv_d)
        for j in range(8):
            pltpu.sync_copy(ones_v.at[pl.ds(j * 128, 128)],
                            accs.at[iv_s.at[j]], add=True)
            pltpu.sync_copy(ones_v.at[pl.ds(j * 128, 128)],
                            accd.at[iv_d.at[j]], add=True)
        return carry

    lax.fori_loop(0, PER_TILE // 1024, body, 0)
    plsc.subcore_barrier()
    pltpu.sync_copy(accs.at[pl.ds(sid * (NP // NSUB), NP // NSUB)],
                    out_h.at[cid, 0, pl.ds(sid * (NP // NSUB), NP // NSUB)])
    pltpu.sync_copy(accd.at[pl.ds(sid * (NP // NSUB), NP // NSUB)],
                    out_h.at[cid, 1, pl.ds(sid * (NP // NSUB), NP // NSUB)])


_CH = 256
_NCH = PER_TILE // _CH


@functools.cache
def _get_agg_kernel():
    return functools.partial(
        pl.kernel,
        out_type=jax.ShapeDtypeStruct((NCORE, NP, D), jnp.float32),
        mesh=_get_mesh(),
        scratch_types=[
            pltpu.VMEM((_CH,), jnp.int32),
            pltpu.VMEM((_CH // 128, 128), jnp.int32),
            pltpu.VMEM((_CH, D), jnp.float32),
            pltpu.VMEM_SHARED((NP, D), jnp.float32),
            pltpu.SemaphoreType.DMA,
        ],
    )(_agg_body)


def _agg_body(y_h, src_h, dst2_h, zrow_h, out_h, sv, dv, rows, acc, sem):
    cid = lax.axis_index("c")
    sid = lax.axis_index("s")
    tid = cid * NSUB + sid
    stripe = NP // NSUB
    pltpu.sync_copy(zrow_h, acc.at[pl.ds(sid * stripe, stripe), :])
    plsc.subcore_barrier()

    def body(i, carry):
        off = tid * PER_TILE + i * _CH
        pltpu.sync_copy(src_h.at[pl.ds(off, _CH)], sv)
        r0 = tid * (PER_TILE // 128) + i * (_CH // 128)
        pltpu.sync_copy(dst2_h.at[pl.ds(r0, _CH // 128), :], dv)
        pltpu.async_copy(y_h.at[sv], rows, sem).wait()
        for j in range(_CH // 128):
            pltpu.sync_copy(rows.at[pl.ds(j * 128, 128), :],
                            acc.at[dv.at[j]], add=True)
        return carry

    lax.fori_loop(0, _NCH, body, 0)
    plsc.subcore_barrier()
    pltpu.sync_copy(acc.at[pl.ds(sid * stripe, stripe), :],
                    out_h.at[cid, pl.ds(sid * stripe, stripe), :])


def _norms(degs, which):
    d = degs[0, which] + degs[1, which]
    return jnp.where(d > 0, lax.rsqrt(d), 0.0)


def _tc_a_body(hp_ref, w1_ref, m1_ref, sumh_ref):
    hp = hp_ref[...]
    m1_ref[...] = jnp.dot(hp, w1_ref[...], preferred_element_type=jnp.float32)
    sumh_ref[...] = jnp.sum(hp, axis=0, keepdims=True)


def _tc_scale_body(m1_ref, degs_ref, y_ref):
    ns = _norms(degs_ref[...], 0)
    y_ref[...] = m1_ref[...] * ns[:, None]


def _bn_relu_masked(parts, nd, g, be):
    a = (parts[0] + parts[1]) * nd[:, None]
    m = jnp.sum(a, axis=0, keepdims=True) / N
    msq = jnp.sum(a * a, axis=0, keepdims=True) / N
    v = msq - m * m
    z = (a - m) * lax.rsqrt(v + 1e-5) * g + be
    z = jnp.maximum(z, 0.0)
    mask = lax.broadcasted_iota(jnp.int32, (NP, 1), 0) < N
    return jnp.where(mask, z, 0.0)


def _tc_b_body(parts_ref, degs_ref, w2_ref, g1_ref, be1_ref, y2_ref, sumz_ref):
    degs = degs_ref[...]
    nd = _norms(degs, 1)
    ns = _norms(degs, 0)
    z = _bn_relu_masked(parts_ref[...], nd, g1_ref[...], be1_ref[...])
    sumz_ref[...] = jnp.sum(z, axis=0, keepdims=True)
    y2_ref[...] = jnp.dot(z * ns[:, None], w2_ref[...],
                          preferred_element_type=jnp.float32)


def _tc_c_body(parts_ref, degs_ref, g2_ref, be2_ref, sumh_ref, sumz1_ref,
               pw0_ref, pb0_ref, pw1_ref, pb1_ref, pw2_ref, pb2_ref, out_ref):
    degs = degs_ref[...]
    nd = _norms(degs, 1)
    z2 = _bn_relu_masked(parts_ref[...], nd, g2_ref[...], be2_ref[...])
    s2 = jnp.sum(z2, axis=0, keepdims=True)
    out_ref[...] = (
        jnp.dot(sumh_ref[...], pw0_ref[...], preferred_element_type=jnp.float32)
        + pb0_ref[...]
        + jnp.dot(sumz1_ref[...], pw1_ref[...], preferred_element_type=jnp.float32)
        + pb1_ref[...]
        + jnp.dot(s2, pw2_ref[...], preferred_element_type=jnp.float32)
        + pb2_ref[...])


_tc_a = pl.pallas_call(
    _tc_a_body,
    out_shape=(jax.ShapeDtypeStruct((NP, D), jnp.float32),
               jax.ShapeDtypeStruct((1, D), jnp.float32)))

_tc_scale = pl.pallas_call(
    _tc_scale_body,
    out_shape=jax.ShapeDtypeStruct((NP, D), jnp.float32))

_tc_b = pl.pallas_call(
    _tc_b_body,
    out_shape=(jax.ShapeDtypeStruct((NP, D), jnp.float32),
               jax.ShapeDtypeStruct((1, D), jnp.float32)))

_tc_c = pl.pallas_call(
    _tc_c_body,
    out_shape=jax.ShapeDtypeStruct((1, D), jnp.float32))


def kernel(h, edge_index, W1, b1, W2, b2, g1, be1, g2, be2,
           PW0, PB0, PW1, PB1, PW2, PB2):
    pad_vals = (N + (jnp.arange(PAD, dtype=jnp.int32) % (NP - N))).astype(jnp.int32)
    srcp = jnp.concatenate([edge_index[0], pad_vals])
    dstp = jnp.concatenate([edge_index[1], pad_vals])
    src2 = srcp.reshape(ER, 128)
    dst2 = dstp.reshape(ER, 128)
    hp = jnp.pad(h, ((0, NP - N), (0, 0)))
    zrow = jnp.zeros((NP // NSUB, D), jnp.float32)
    zero1 = jnp.zeros((NP,), jnp.float32)
    ones1k = jnp.ones((1024,), jnp.float32)

    degs = _get_deg_kernel()(src2, dst2, ones1k, zero1)
    m1, sumh = _tc_a(hp, W1)
    y1 = _tc_scale(m1, degs)
    parts1 = _get_agg_kernel()(y1, srcp, dst2, zrow)
    y2, sumz1 = _tc_b(parts1, degs, W2, g1, be1)
    parts2 = _get_agg_kernel()(y2, srcp, dst2, zrow)
    return _tc_c(parts2, degs, g2, be2, sumh, sumz1,
                 PW0, PB0, PW1, PB1, PW2, PB2)


# speedup vs baseline: 1.1244x; 1.0782x over previous
import functools

import jax
import jax.numpy as jnp
from jax import lax
from jax.experimental import pallas as pl
from jax.experimental.pallas import tpu as pltpu
from jax.experimental.pallas import tpu_sc as plsc

N = 10000
E = 320000
D = 128
NP = 10240
NCORE = 2
NSUB = 16
TILES = NCORE * NSUB
PER_TILE = 10240
EP = TILES * PER_TILE
PAD = EP - E
ER = EP // 128

@functools.cache
def _get_mesh():
    return plsc.VectorSubcoreMesh(
        core_axis_name="c", subcore_axis_name="s",
        num_cores=NCORE, num_subcores=NSUB)


@functools.cache
def _get_deg_kernel():
    return functools.partial(
        pl.kernel,
        out_type=jax.ShapeDtypeStruct((NCORE, 2, NP), jnp.float32),
        mesh=_get_mesh(),
        scratch_types=[
            pltpu.VMEM((8, 128), jnp.int32),
            pltpu.VMEM((8, 128), jnp.int32),
            pltpu.VMEM((1024,), jnp.float32),
            pltpu.VMEM_SHARED((NP,), jnp.float32),
            pltpu.VMEM_SHARED((NP,), jnp.float32),
        ],
    )(_deg_body)


def _deg_body(src2_h, dst2_h, ones_h, zero1_h, out_h,
              iv_s, iv_d, ones_v, accs, accd):
    cid = lax.axis_index("c")
    sid = lax.axis_index("s")
    tid = cid * NSUB + sid
    pltpu.sync_copy(ones_h, ones_v)

    @pl.when(sid == 0)
    def _():
        pltpu.sync_copy(zero1_h, accs)

    @pl.when(sid == 1)
    def _():
        pltpu.sync_copy(zero1_h, accd)

    plsc.subcore_barrier()

    def body(i, carry):
        r0 = tid * (PER_TILE // 128) + i * 8
        pltpu.sync_copy(src2_h.at[pl.ds(r0, 8), :], iv_s)
        pltpu.sync_copy(dst2_h.at[pl.ds(r0, 8), :], iv_d)
        for j in range(8):
            pltpu.sync_copy(ones_v.at[pl.ds(j * 128, 128)],
                            accs.at[iv_s.at[j]], add=True)
            pltpu.sync_copy(ones_v.at[pl.ds(j * 128, 128)],
                            accd.at[iv_d.at[j]], add=True)
        return carry

    lax.fori_loop(0, PER_TILE // 1024, body, 0)
    plsc.subcore_barrier()
    pltpu.sync_copy(accs.at[pl.ds(sid * (NP // NSUB), NP // NSUB)],
                    out_h.at[cid, 0, pl.ds(sid * (NP // NSUB), NP // NSUB)])
    pltpu.sync_copy(accd.at[pl.ds(sid * (NP // NSUB), NP // NSUB)],
                    out_h.at[cid, 1, pl.ds(sid * (NP // NSUB), NP // NSUB)])


_CH = 256
_NCH = PER_TILE // _CH


@functools.cache
def _get_agg_kernel():
    return functools.partial(
        pl.kernel,
        out_type=jax.ShapeDtypeStruct((NCORE, NP, D), jnp.float32),
        mesh=_get_mesh(),
        scratch_types=[
            pltpu.VMEM((PER_TILE,), jnp.int32),
            pltpu.VMEM((_CH // 128, 128), jnp.int32),
            pltpu.VMEM((_CH, D), jnp.float32),
            pltpu.VMEM_SHARED((NP, D), jnp.float32),
            pltpu.SemaphoreType.DMA,
        ],
    )(_agg_body)


def _agg_body(y_h, src_h, dst2_h, zrow_h, out_h, sv_all, dv, rows, acc, sem):
    cid = lax.axis_index("c")
    sid = lax.axis_index("s")
    tid = cid * NSUB + sid
    stripe = NP // NSUB
    pltpu.sync_copy(src_h.at[pl.ds(tid * PER_TILE, PER_TILE)], sv_all)
    pltpu.sync_copy(zrow_h, acc.at[pl.ds(sid * stripe, stripe), :])
    plsc.subcore_barrier()

    def body(i, carry):
        r0 = tid * (PER_TILE // 128) + i * (_CH // 128)
        pltpu.sync_copy(dst2_h.at[pl.ds(r0, _CH // 128), :], dv)
        pltpu.async_copy(y_h.at[sv_all.at[pl.ds(i * _CH, _CH)]],
                         rows, sem).wait()
        for j in range(_CH // 128):
            pltpu.sync_copy(rows.at[pl.ds(j * 128, 128), :],
                            acc.at[dv.at[j]], add=True)
        return carry

    lax.fori_loop(0, _NCH, body, 0)
    plsc.subcore_barrier()
    pltpu.sync_copy(acc.at[pl.ds(sid * stripe, stripe), :],
                    out_h.at[cid, pl.ds(sid * stripe, stripe), :])


def _norms(degs, which):
    d = degs[0, which] + degs[1, which]
    return jnp.where(d > 0, lax.rsqrt(d), 0.0)


def _tc_a_body(hp_ref, w1_ref, m1_ref, sumh_ref):
    hp = hp_ref[...]
    m1_ref[...] = jnp.dot(hp, w1_ref[...], preferred_element_type=jnp.float32)
    sumh_ref[...] = jnp.sum(hp, axis=0, keepdims=True)


def _tc_scale_body(m1_ref, degs_ref, y_ref):
    ns = _norms(degs_ref[...], 0)
    y_ref[...] = m1_ref[...] * ns[:, None]


def _bn_relu_masked(parts, nd, g, be):
    a = (parts[0] + parts[1]) * nd[:, None]
    m = jnp.sum(a, axis=0, keepdims=True) / N
    msq = jnp.sum(a * a, axis=0, keepdims=True) / N
    v = msq - m * m
    z = (a - m) * lax.rsqrt(v + 1e-5) * g + be
    z = jnp.maximum(z, 0.0)
    mask = lax.broadcasted_iota(jnp.int32, (NP, 1), 0) < N
    return jnp.where(mask, z, 0.0)


def _tc_b_body(parts_ref, degs_ref, w2_ref, g1_ref, be1_ref, y2_ref, sumz_ref):
    degs = degs_ref[...]
    nd = _norms(degs, 1)
    ns = _norms(degs, 0)
    z = _bn_relu_masked(parts_ref[...], nd, g1_ref[...], be1_ref[...])
    sumz_ref[...] = jnp.sum(z, axis=0, keepdims=True)
    y2_ref[...] = jnp.dot(z * ns[:, None], w2_ref[...],
                          preferred_element_type=jnp.float32)


def _tc_c_body(parts_ref, degs_ref, g2_ref, be2_ref, sumh_ref, sumz1_ref,
               pw0_ref, pb0_ref, pw1_ref, pb1_ref, pw2_ref, pb2_ref, out_ref):
    degs = degs_ref[...]
    nd = _norms(degs, 1)
    z2 = _bn_relu_masked(parts_ref[...], nd, g2_ref[...], be2_ref[...])
    s2 = jnp.sum(z2, axis=0, keepdims=True)
    out_ref[...] = (
        jnp.dot(sumh_ref[...], pw0_ref[...], preferred_element_type=jnp.float32)
        + pb0_ref[...]
        + jnp.dot(sumz1_ref[...], pw1_ref[...], preferred_element_type=jnp.float32)
        + pb1_ref[...]
        + jnp.dot(s2, pw2_ref[...], preferred_element_type=jnp.float32)
        + pb2_ref[...])


_tc_a = pl.pallas_call(
    _tc_a_body,
    out_shape=(jax.ShapeDtypeStruct((NP, D), jnp.float32),
               jax.ShapeDtypeStruct((1, D), jnp.float32)))

_tc_scale = pl.pallas_call(
    _tc_scale_body,
    out_shape=jax.ShapeDtypeStruct((NP, D), jnp.float32))

_tc_b = pl.pallas_call(
    _tc_b_body,
    out_shape=(jax.ShapeDtypeStruct((NP, D), jnp.float32),
               jax.ShapeDtypeStruct((1, D), jnp.float32)))

_tc_c = pl.pallas_call(
    _tc_c_body,
    out_shape=jax.ShapeDtypeStruct((1, D), jnp.float32))


def kernel(h, edge_index, W1, b1, W2, b2, g1, be1, g2, be2,
           PW0, PB0, PW1, PB1, PW2, PB2):
    pad_vals = (N + (jnp.arange(PAD, dtype=jnp.int32) % (NP - N))).astype(jnp.int32)
    srcp = jnp.concatenate([edge_index[0], pad_vals])
    dstp = jnp.concatenate([edge_index[1], pad_vals])
    src2 = srcp.reshape(ER, 128)
    dst2 = dstp.reshape(ER, 128)
    hp = jnp.pad(h, ((0, NP - N), (0, 0)))
    zrow = jnp.zeros((NP // NSUB, D), jnp.float32)
    zero1 = jnp.zeros((NP,), jnp.float32)
    ones1k = jnp.ones((1024,), jnp.float32)

    degs = _get_deg_kernel()(src2, dst2, ones1k, zero1)
    m1, sumh = _tc_a(hp, W1)
    y1 = _tc_scale(m1, degs)
    parts1 = _get_agg_kernel()(y1, srcp, dst2, zrow)
    y2, sumz1 = _tc_b(parts1, degs, W2, g1, be1)
    parts2 = _get_agg_kernel()(y2, srcp, dst2, zrow)
    return _tc_c(parts2, degs, g2, be2, sumh, sumz1,
                 PW0, PB0, PW1, PB1, PW2, PB2)


# speedup vs baseline: 1.1430x; 1.0166x over previous
import functools

import jax
import jax.numpy as jnp
from jax import lax
from jax.experimental import pallas as pl
from jax.experimental.pallas import tpu as pltpu
from jax.experimental.pallas import tpu_sc as plsc

N = 10000
E = 320000
D = 128
NP = 10240
NCORE = 2
NSUB = 16
TILES = NCORE * NSUB
PER_TILE = 10240
EP = TILES * PER_TILE
PAD = EP - E
ER = EP // 128

@functools.cache
def _get_mesh():
    return plsc.VectorSubcoreMesh(
        core_axis_name="c", subcore_axis_name="s",
        num_cores=NCORE, num_subcores=NSUB)


@functools.cache
def _get_deg_kernel():
    return functools.partial(
        pl.kernel,
        out_type=jax.ShapeDtypeStruct((NCORE, 2, NP), jnp.float32),
        mesh=_get_mesh(),
        scratch_types=[
            pltpu.VMEM((PER_TILE // 128, 128), jnp.int32),
            pltpu.VMEM((PER_TILE // 128, 128), jnp.int32),
            pltpu.VMEM((128,), jnp.float32),
            pltpu.VMEM_SHARED((NP,), jnp.float32),
            pltpu.VMEM_SHARED((NP,), jnp.float32),
            pltpu.SemaphoreType.DMA,
            pltpu.SemaphoreType.DMA,
        ],
    )(_deg_body)


def _deg_body(src2_h, dst2_h, ones_h, zero1_h, out_h,
              iv_s, iv_d, ones_v, accs, accd, sem_s, sem_d):
    cid = lax.axis_index("c")
    sid = lax.axis_index("s")
    tid = cid * NSUB + sid
    pltpu.sync_copy(ones_h, ones_v)
    pltpu.sync_copy(src2_h.at[pl.ds(tid * (PER_TILE // 128),
                                    PER_TILE // 128), :], iv_s)
    pltpu.sync_copy(dst2_h.at[pl.ds(tid * (PER_TILE // 128),
                                    PER_TILE // 128), :], iv_d)

    @pl.when(sid == 0)
    def _():
        pltpu.sync_copy(zero1_h, accs)

    @pl.when(sid == 1)
    def _():
        pltpu.sync_copy(zero1_h, accd)

    plsc.subcore_barrier()

    _B = 8

    def body(i, carry):
        for j in range(_B):
            r = i * _B + j
            pltpu.async_copy(ones_v, accs.at[iv_s.at[r]], sem_s, add=True)
            pltpu.async_copy(ones_v, accd.at[iv_d.at[r]], sem_d, add=True)
        for j in range(_B):
            r = i * _B + j
            pltpu.make_async_copy(ones_v, accs.at[iv_s.at[r]], sem_s).wait()
            pltpu.make_async_copy(ones_v, accd.at[iv_d.at[r]], sem_d).wait()
        return carry

    lax.fori_loop(0, PER_TILE // 128 // _B, body, 0)
    plsc.subcore_barrier()
    pltpu.sync_copy(accs.at[pl.ds(sid * (NP // NSUB), NP // NSUB)],
                    out_h.at[cid, 0, pl.ds(sid * (NP // NSUB), NP // NSUB)])
    pltpu.sync_copy(accd.at[pl.ds(sid * (NP // NSUB), NP // NSUB)],
                    out_h.at[cid, 1, pl.ds(sid * (NP // NSUB), NP // NSUB)])


_CH = 128
_NCH = PER_TILE // _CH


@functools.cache
def _get_agg_kernel():
    return functools.partial(
        pl.kernel,
        out_type=jax.ShapeDtypeStruct((NCORE, NP, D), jnp.float32),
        mesh=_get_mesh(),
        scratch_types=[
            pltpu.VMEM((PER_TILE,), jnp.int32),
            pltpu.VMEM((PER_TILE // 128, 128), jnp.int32),
            pltpu.VMEM((_CH, D), jnp.float32),
            pltpu.VMEM_SHARED((NP, D), jnp.float32),
            pltpu.SemaphoreType.DMA,
        ],
    )(_agg_body)


def _agg_body(y_h, src_h, dst2_h, zrow_h, out_h, sv_all, dv_all, rows, acc,
              sem):
    cid = lax.axis_index("c")
    sid = lax.axis_index("s")
    tid = cid * NSUB + sid
    stripe = NP // NSUB
    pltpu.sync_copy(src_h.at[pl.ds(tid * PER_TILE, PER_TILE)], sv_all)
    pltpu.sync_copy(
        dst2_h.at[pl.ds(tid * (PER_TILE // 128), PER_TILE // 128), :], dv_all)
    pltpu.sync_copy(zrow_h, acc.at[pl.ds(sid * stripe, stripe), :])
    plsc.subcore_barrier()

    def body(i, carry):
        pltpu.async_copy(y_h.at[sv_all.at[pl.ds(i * _CH, _CH)]],
                         rows, sem).wait()
        for j in range(_CH // 128):
            pltpu.sync_copy(rows.at[pl.ds(j * 128, 128), :],
                            acc.at[dv_all.at[i * (_CH // 128) + j]], add=True)
        return carry

    lax.fori_loop(0, _NCH, body, 0)
    plsc.subcore_barrier()
    pltpu.sync_copy(acc.at[pl.ds(sid * stripe, stripe), :],
                    out_h.at[cid, pl.ds(sid * stripe, stripe), :])


def _norms(degs, which):
    d = degs[0, which] + degs[1, which]
    return jnp.where(d > 0, lax.rsqrt(d), 0.0)


def _tc_a_body(hp_ref, w1_ref, m1_ref, sumh_ref):
    hp = hp_ref[...]
    m1_ref[...] = jnp.dot(hp, w1_ref[...], preferred_element_type=jnp.float32)
    sumh_ref[...] = jnp.sum(hp, axis=0, keepdims=True)


def _tc_scale_body(m1_ref, degs_ref, y_ref):
    ns = _norms(degs_ref[...], 0)
    y_ref[...] = m1_ref[...] * ns[:, None]


def _bn_relu_masked(parts, nd, g, be):
    a = (parts[0] + parts[1]) * nd[:, None]
    m = jnp.sum(a, axis=0, keepdims=True) / N
    msq = jnp.sum(a * a, axis=0, keepdims=True) / N
    v = msq - m * m
    z = (a - m) * lax.rsqrt(v + 1e-5) * g + be
    z = jnp.maximum(z, 0.0)
    mask = lax.broadcasted_iota(jnp.int32, (NP, 1), 0) < N
    return jnp.where(mask, z, 0.0)


def _tc_b_body(parts_ref, degs_ref, w2_ref, g1_ref, be1_ref, y2_ref, sumz_ref):
    degs = degs_ref[...]
    nd = _norms(degs, 1)
    ns = _norms(degs, 0)
    z = _bn_relu_masked(parts_ref[...], nd, g1_ref[...], be1_ref[...])
    sumz_ref[...] = jnp.sum(z, axis=0, keepdims=True)
    y2_ref[...] = jnp.dot(z * ns[:, None], w2_ref[...],
                          preferred_element_type=jnp.float32)


def _tc_c_body(parts_ref, degs_ref, g2_ref, be2_ref, sumh_ref, sumz1_ref,
               pw0_ref, pb0_ref, pw1_ref, pb1_ref, pw2_ref, pb2_ref, out_ref):
    degs = degs_ref[...]
    nd = _norms(degs, 1)
    z2 = _bn_relu_masked(parts_ref[...], nd, g2_ref[...], be2_ref[...])
    s2 = jnp.sum(z2, axis=0, keepdims=True)
    out_ref[...] = (
        jnp.dot(sumh_ref[...], pw0_ref[...], preferred_element_type=jnp.float32)
        + pb0_ref[...]
        + jnp.dot(sumz1_ref[...], pw1_ref[...], preferred_element_type=jnp.float32)
        + pb1_ref[...]
        + jnp.dot(s2, pw2_ref[...], preferred_element_type=jnp.float32)
        + pb2_ref[...])


_tc_a = pl.pallas_call(
    _tc_a_body,
    out_shape=(jax.ShapeDtypeStruct((NP, D), jnp.float32),
               jax.ShapeDtypeStruct((1, D), jnp.float32)))

_tc_scale = pl.pallas_call(
    _tc_scale_body,
    out_shape=jax.ShapeDtypeStruct((NP, D), jnp.float32))

_tc_b = pl.pallas_call(
    _tc_b_body,
    out_shape=(jax.ShapeDtypeStruct((NP, D), jnp.float32),
               jax.ShapeDtypeStruct((1, D), jnp.float32)))

_tc_c = pl.pallas_call(
    _tc_c_body,
    out_shape=jax.ShapeDtypeStruct((1, D), jnp.float32))


def kernel(h, edge_index, W1, b1, W2, b2, g1, be1, g2, be2,
           PW0, PB0, PW1, PB1, PW2, PB2):
    pad_vals = (N + (jnp.arange(PAD, dtype=jnp.int32) % (NP - N))).astype(jnp.int32)
    srcp = jnp.concatenate([edge_index[0], pad_vals])
    dstp = jnp.concatenate([edge_index[1], pad_vals])
    src2 = srcp.reshape(ER, 128)
    dst2 = dstp.reshape(ER, 128)
    hp = jnp.pad(h, ((0, NP - N), (0, 0)))
    zrow = jnp.zeros((NP // NSUB, D), jnp.float32)
    zero1 = jnp.zeros((NP,), jnp.float32)
    ones128 = jnp.ones((128,), jnp.float32)

    degs = _get_deg_kernel()(src2, dst2, ones128, zero1)
    m1, sumh = _tc_a(hp, W1)
    y1 = _tc_scale(m1, degs)
    parts1 = _get_agg_kernel()(y1, srcp, dst2, zrow)
    y2, sumz1 = _tc_b(parts1, degs, W2, g1, be1)
    parts2 = _get_agg_kernel()(y2, srcp, dst2, zrow)
    return _tc_c(parts2, degs, g2, be2, sumh, sumz1,
                 PW0, PB0, PW1, PB1, PW2, PB2)


# speedup vs baseline: 1.4651x; 1.2818x over previous
import functools

import jax
import jax.numpy as jnp
from jax import lax
from jax.experimental import pallas as pl
from jax.experimental.pallas import tpu as pltpu
from jax.experimental.pallas import tpu_sc as plsc

N = 10000
E = 320000
D = 128
NP = 10240
NCORE = 2
NSUB = 16
TILES = NCORE * NSUB
PER_TILE = 10240
EP = TILES * PER_TILE
PAD = EP - E
ER = EP // 128

@functools.cache
def _get_mesh():
    return plsc.VectorSubcoreMesh(
        core_axis_name="c", subcore_axis_name="s",
        num_cores=NCORE, num_subcores=NSUB)


@functools.cache
def _get_deg_kernel():
    return functools.partial(
        pl.kernel,
        out_type=jax.ShapeDtypeStruct((NCORE, 2, NP), jnp.float32),
        mesh=_get_mesh(),
        scratch_types=[
            pltpu.VMEM((PER_TILE // 128, 128), jnp.int32),
            pltpu.VMEM((PER_TILE // 128, 128), jnp.int32),
            pltpu.VMEM((128,), jnp.float32),
            pltpu.VMEM_SHARED((NP,), jnp.float32),
            pltpu.VMEM_SHARED((NP,), jnp.float32),
            pltpu.SemaphoreType.DMA,
            pltpu.SemaphoreType.DMA,
        ],
    )(_deg_body)


def _deg_body(src2_h, dst2_h, ones_h, zero1_h, out_h,
              iv_s, iv_d, ones_v, accs, accd, sem_s, sem_d):
    cid = lax.axis_index("c")
    sid = lax.axis_index("s")
    tid = cid * NSUB + sid
    pltpu.sync_copy(ones_h, ones_v)
    pltpu.sync_copy(src2_h.at[pl.ds(tid * (PER_TILE // 128),
                                    PER_TILE // 128), :], iv_s)
    pltpu.sync_copy(dst2_h.at[pl.ds(tid * (PER_TILE // 128),
                                    PER_TILE // 128), :], iv_d)

    @pl.when(sid == 0)
    def _():
        pltpu.sync_copy(zero1_h, accs)

    @pl.when(sid == 1)
    def _():
        pltpu.sync_copy(zero1_h, accd)

    plsc.subcore_barrier()

    _B = 8

    def body(i, carry):
        for j in range(_B):
            r = i * _B + j
            pltpu.async_copy(ones_v, accs.at[iv_s.at[r]], sem_s, add=True)
            pltpu.async_copy(ones_v, accd.at[iv_d.at[r]], sem_d, add=True)
        for j in range(_B):
            r = i * _B + j
            pltpu.make_async_copy(ones_v, accs.at[iv_s.at[r]], sem_s).wait()
            pltpu.make_async_copy(ones_v, accd.at[iv_d.at[r]], sem_d).wait()
        return carry

    lax.fori_loop(0, PER_TILE // 128 // _B, body, 0)
    plsc.subcore_barrier()
    pltpu.sync_copy(accs.at[pl.ds(sid * (NP // NSUB), NP // NSUB)],
                    out_h.at[cid, 0, pl.ds(sid * (NP // NSUB), NP // NSUB)])
    pltpu.sync_copy(accd.at[pl.ds(sid * (NP // NSUB), NP // NSUB)],
                    out_h.at[cid, 1, pl.ds(sid * (NP // NSUB), NP // NSUB)])


_CH = 128
_NCH = PER_TILE // _CH


@functools.cache
def _get_agg_kernel():
    return functools.partial(
        pl.kernel,
        out_type=jax.ShapeDtypeStruct((NCORE, NP, D), jnp.float32),
        mesh=_get_mesh(),
        scratch_types=[
            pltpu.VMEM((PER_TILE,), jnp.int32),
            pltpu.VMEM((_CH,), jnp.int32),
            pltpu.VMEM((_CH,), jnp.int32),
            pltpu.VMEM((1, _CH), jnp.int32),
            pltpu.VMEM((1, _CH), jnp.int32),
            pltpu.VMEM((_CH, D), jnp.float32),
            pltpu.VMEM((_CH, D), jnp.float32),
            pltpu.VMEM_SHARED((NP, D), jnp.float32),
            pltpu.SemaphoreType.DMA,
            pltpu.SemaphoreType.DMA,
        ],
    )(_agg_body)


def _agg_body(y_h, epk_h, zrow_h, out_h, epk_v, sv0, sv1, dv0, dv1,
              rows0, rows1, acc, gsem0, gsem1):
    cid = lax.axis_index("c")
    sid = lax.axis_index("s")
    tid = cid * NSUB + sid
    stripe = NP // NSUB
    pltpu.sync_copy(epk_h.at[pl.ds(tid * PER_TILE, PER_TILE)], epk_v)
    pltpu.sync_copy(zrow_h, acc.at[pl.ds(sid * stripe, stripe), :])
    plsc.subcore_barrier()

    def unpack(i, sv, dv):
        for k in range(_CH // 16):
            w = epk_v[pl.ds(i * _CH + k * 16, 16)]
            sv[pl.ds(k * 16, 16)] = lax.shift_right_logical(w, 16)
            dv[0, pl.ds(k * 16, 16)] = lax.bitwise_and(w, 0xFFFF)

    unpack(0, sv0, dv0)
    pltpu.async_copy(y_h.at[sv0], rows0, gsem0)
    nsteps = _NCH // 2

    def step(s, carry):
        i0 = s * 2
        unpack(i0 + 1, sv1, dv1)
        pltpu.make_async_copy(y_h.at[sv0], rows0, gsem0).wait()
        pltpu.async_copy(y_h.at[sv1], rows1, gsem1)
        pltpu.sync_copy(rows0, acc.at[dv0.at[0]], add=True)

        @pl.when(s + 1 < nsteps)
        def _():
            unpack(i0 + 2, sv0, dv0)

        pltpu.make_async_copy(y_h.at[sv1], rows1, gsem1).wait()

        @pl.when(s + 1 < nsteps)
        def _():
            pltpu.async_copy(y_h.at[sv0], rows0, gsem0)

        pltpu.sync_copy(rows1, acc.at[dv1.at[0]], add=True)
        return carry

    lax.fori_loop(0, nsteps, step, 0)
    plsc.subcore_barrier()
    pltpu.sync_copy(acc.at[pl.ds(sid * stripe, stripe), :],
                    out_h.at[cid, pl.ds(sid * stripe, stripe), :])


def _norms(degs, which):
    d = degs[0, which] + degs[1, which]
    return jnp.where(d > 0, lax.rsqrt(d), 0.0)


def _tc_a_body(hp_ref, w1_ref, m1_ref, sumh_ref):
    hp = hp_ref[...]
    m1_ref[...] = jnp.dot(hp, w1_ref[...], preferred_element_type=jnp.float32)
    sumh_ref[...] = jnp.sum(hp, axis=0, keepdims=True)


def _tc_scale_body(m1_ref, degs_ref, y_ref):
    ns = _norms(degs_ref[...], 0)
    y_ref[...] = m1_ref[...] * ns[:, None]


def _bn_relu_masked(parts, nd, g, be):
    a = (parts[0] + parts[1]) * nd[:, None]
    m = jnp.sum(a, axis=0, keepdims=True) / N
    msq = jnp.sum(a * a, axis=0, keepdims=True) / N
    v = msq - m * m
    z = (a - m) * lax.rsqrt(v + 1e-5) * g + be
    z = jnp.maximum(z, 0.0)
    mask = lax.broadcasted_iota(jnp.int32, (NP, 1), 0) < N
    return jnp.where(mask, z, 0.0)


def _tc_b_body(parts_ref, degs_ref, w2_ref, g1_ref, be1_ref, y2_ref, sumz_ref):
    degs = degs_ref[...]
    nd = _norms(degs, 1)
    ns = _norms(degs, 0)
    z = _bn_relu_masked(parts_ref[...], nd, g1_ref[...], be1_ref[...])
    sumz_ref[...] = jnp.sum(z, axis=0, keepdims=True)
    y2_ref[...] = jnp.dot(z * ns[:, None], w2_ref[...],
                          preferred_element_type=jnp.float32)


def _tc_c_body(parts_ref, degs_ref, g2_ref, be2_ref, sumh_ref, sumz1_ref,
               pw0_ref, pb0_ref, pw1_ref, pb1_ref, pw2_ref, pb2_ref, out_ref):
    degs = degs_ref[...]
    nd = _norms(degs, 1)
    z2 = _bn_relu_masked(parts_ref[...], nd, g2_ref[...], be2_ref[...])
    s2 = jnp.sum(z2, axis=0, keepdims=True)
    out_ref[...] = (
        jnp.dot(sumh_ref[...], pw0_ref[...], preferred_element_type=jnp.float32)
        + pb0_ref[...]
        + jnp.dot(sumz1_ref[...], pw1_ref[...], preferred_element_type=jnp.float32)
        + pb1_ref[...]
        + jnp.dot(s2, pw2_ref[...], preferred_element_type=jnp.float32)
        + pb2_ref[...])


_tc_a = pl.pallas_call(
    _tc_a_body,
    out_shape=(jax.ShapeDtypeStruct((NP, D), jnp.float32),
               jax.ShapeDtypeStruct((1, D), jnp.float32)))

_tc_scale = pl.pallas_call(
    _tc_scale_body,
    out_shape=jax.ShapeDtypeStruct((NP, D), jnp.float32))

_tc_b = pl.pallas_call(
    _tc_b_body,
    out_shape=(jax.ShapeDtypeStruct((NP, D), jnp.float32),
               jax.ShapeDtypeStruct((1, D), jnp.float32)))

_tc_c = pl.pallas_call(
    _tc_c_body,
    out_shape=jax.ShapeDtypeStruct((1, D), jnp.float32))


def kernel(h, edge_index, W1, b1, W2, b2, g1, be1, g2, be2,
           PW0, PB0, PW1, PB1, PW2, PB2):
    pad_vals = (N + (jnp.arange(PAD, dtype=jnp.int32) % (NP - N))).astype(jnp.int32)
    srcp = jnp.concatenate([edge_index[0], pad_vals])
    dstp = jnp.concatenate([edge_index[1], pad_vals])
    src2 = srcp.reshape(ER, 128)
    dst2 = dstp.reshape(ER, 128)
    hp = jnp.pad(h, ((0, NP - N), (0, 0)))
    zrow = jnp.zeros((NP // NSUB, D), jnp.float32)
    zero1 = jnp.zeros((NP,), jnp.float32)
    ones128 = jnp.ones((128,), jnp.float32)

    epk = jnp.bitwise_or(jnp.left_shift(srcp, 16), dstp)

    degs = _get_deg_kernel()(src2, dst2, ones128, zero1)
    m1, sumh = _tc_a(hp, W1)
    y1 = _tc_scale(m1, degs)
    parts1 = _get_agg_kernel()(y1, epk, zrow)
    y2, sumz1 = _tc_b(parts1, degs, W2, g1, be1)
    parts2 = _get_agg_kernel()(y2, epk, zrow)
    return _tc_c(parts2, degs, g2, be2, sumh, sumz1,
                 PW0, PB0, PW1, PB1, PW2, PB2)


# speedup vs baseline: 1.4837x; 1.0127x over previous
import functools

import jax
import jax.numpy as jnp
from jax import lax
from jax.experimental import pallas as pl
from jax.experimental.pallas import tpu as pltpu
from jax.experimental.pallas import tpu_sc as plsc

N = 10000
E = 320000
D = 128
NP = 10240
NCORE = 2
NSUB = 16
TILES = NCORE * NSUB
PER_TILE = 10240
EP = TILES * PER_TILE
PAD = EP - E
ER = EP // 128

@functools.cache
def _get_mesh():
    return plsc.VectorSubcoreMesh(
        core_axis_name="c", subcore_axis_name="s",
        num_cores=NCORE, num_subcores=NSUB)


@functools.cache
def _get_deg_kernel():
    return functools.partial(
        pl.kernel,
        out_type=jax.ShapeDtypeStruct((NCORE, 2, NP), jnp.float32),
        mesh=_get_mesh(),
        scratch_types=[
            pltpu.VMEM((PER_TILE // 128, 128), jnp.int32),
            pltpu.VMEM((PER_TILE // 128, 128), jnp.int32),
            pltpu.VMEM((128,), jnp.float32),
            pltpu.VMEM_SHARED((NP,), jnp.float32),
            pltpu.VMEM_SHARED((NP,), jnp.float32),
            pltpu.SemaphoreType.DMA,
            pltpu.SemaphoreType.DMA,
        ],
    )(_deg_body)


def _deg_body(src2_h, dst2_h, ones_h, zero1_h, out_h,
              iv_s, iv_d, ones_v, accs, accd, sem_s, sem_d):
    cid = lax.axis_index("c")
    sid = lax.axis_index("s")
    tid = cid * NSUB + sid
    pltpu.sync_copy(ones_h, ones_v)
    pltpu.sync_copy(src2_h.at[pl.ds(tid * (PER_TILE // 128),
                                    PER_TILE // 128), :], iv_s)
    pltpu.sync_copy(dst2_h.at[pl.ds(tid * (PER_TILE // 128),
                                    PER_TILE // 128), :], iv_d)

    @pl.when(sid == 0)
    def _():
        pltpu.sync_copy(zero1_h, accs)

    @pl.when(sid == 1)
    def _():
        pltpu.sync_copy(zero1_h, accd)

    plsc.subcore_barrier()

    _B = 8

    def body(i, carry):
        for j in range(_B):
            r = i * _B + j
            pltpu.async_copy(ones_v, accs.at[iv_s.at[r]], sem_s, add=True)
            pltpu.async_copy(ones_v, accd.at[iv_d.at[r]], sem_d, add=True)
        for j in range(_B):
            r = i * _B + j
            pltpu.make_async_copy(ones_v, accs.at[iv_s.at[r]], sem_s).wait()
            pltpu.make_async_copy(ones_v, accd.at[iv_d.at[r]], sem_d).wait()
        return carry

    lax.fori_loop(0, PER_TILE // 128 // _B, body, 0)
    plsc.subcore_barrier()
    pltpu.sync_copy(accs.at[pl.ds(sid * (NP // NSUB), NP // NSUB)],
                    out_h.at[cid, 0, pl.ds(sid * (NP // NSUB), NP // NSUB)])
    pltpu.sync_copy(accd.at[pl.ds(sid * (NP // NSUB), NP // NSUB)],
                    out_h.at[cid, 1, pl.ds(sid * (NP // NSUB), NP // NSUB)])


_CH = 128
_NCH = PER_TILE // _CH


@functools.cache
def _get_agg_kernel():
    return functools.partial(
        pl.kernel,
        out_type=jax.ShapeDtypeStruct((NCORE, NP, D), jnp.float32),
        mesh=_get_mesh(),
        scratch_types=[
            pltpu.VMEM((PER_TILE,), jnp.int32),
            pltpu.VMEM((_CH,), jnp.int32),
            pltpu.VMEM((_CH,), jnp.int32),
            pltpu.VMEM((1, _CH), jnp.int32),
            pltpu.VMEM((1, _CH), jnp.int32),
            pltpu.VMEM((_CH, D), jnp.float32),
            pltpu.VMEM((_CH, D), jnp.float32),
            pltpu.VMEM_SHARED((NP, D), jnp.float32),
            pltpu.SemaphoreType.DMA,
            pltpu.SemaphoreType.DMA,
        ],
    )(_agg_body)


def _agg_body(y_h, epk_h, zrow_h, out_h, epk_v, sv0, sv1, dv0, dv1,
              rows0, rows1, acc, gsem0, gsem1):
    cid = lax.axis_index("c")
    sid = lax.axis_index("s")
    tid = cid * NSUB + sid
    stripe = NP // NSUB
    pltpu.sync_copy(epk_h.at[pl.ds(tid * PER_TILE, PER_TILE)], epk_v)
    pltpu.sync_copy(zrow_h, acc.at[pl.ds(sid * stripe, stripe), :])
    plsc.subcore_barrier()

    def unpack(i, sv, dv):
        for k in range(_CH // 16):
            w = epk_v[pl.ds(i * _CH + k * 16, 16)]
            sv[pl.ds(k * 16, 16)] = lax.shift_right_logical(w, 16)
            dv[0, pl.ds(k * 16, 16)] = lax.bitwise_and(w, 0xFFFF)

    unpack(0, sv0, dv0)
    pltpu.async_copy(y_h.at[sv0], rows0, gsem0)
    nsteps = _NCH // 2

    def step(s, carry):
        i0 = s * 2
        unpack(i0 + 1, sv1, dv1)
        pltpu.make_async_copy(y_h.at[sv0], rows0, gsem0).wait()
        pltpu.async_copy(y_h.at[sv1], rows1, gsem1)
        pltpu.sync_copy(rows0, acc.at[dv0.at[0]], add=True)

        @pl.when(s + 1 < nsteps)
        def _():
            unpack(i0 + 2, sv0, dv0)

        pltpu.make_async_copy(y_h.at[sv1], rows1, gsem1).wait()

        @pl.when(s + 1 < nsteps)
        def _():
            pltpu.async_copy(y_h.at[sv0], rows0, gsem0)

        pltpu.sync_copy(rows1, acc.at[dv1.at[0]], add=True)
        return carry

    lax.fori_loop(0, nsteps, step, 0)
    plsc.subcore_barrier()
    pltpu.sync_copy(acc.at[pl.ds(sid * stripe, stripe), :],
                    out_h.at[cid, pl.ds(sid * stripe, stripe), :])


def _norms(degs, which):
    d = degs[0, which] + degs[1, which]
    return jnp.where(d > 0, lax.rsqrt(d), 0.0)


def _tc_a_body(h_ref, w1_ref, m1_ref, sumh_ref):
    h = h_ref[...]
    m1_ref[pl.ds(0, N), :] = jnp.dot(h, w1_ref[...],
                                     preferred_element_type=jnp.float32)
    m1_ref[pl.ds(N, NP - N), :] = jnp.zeros((NP - N, D), jnp.float32)
    sumh_ref[...] = jnp.sum(h, axis=0, keepdims=True)


def _tc_scale_body(m1_ref, degs_ref, y_ref):
    ns = _norms(degs_ref[...], 0)
    y_ref[...] = m1_ref[...] * ns[:, None]


def _bn_relu_masked(parts, nd, g, be):
    a = (parts[0] + parts[1]) * nd[:, None]
    m = jnp.sum(a, axis=0, keepdims=True) / N
    msq = jnp.sum(a * a, axis=0, keepdims=True) / N
    v = msq - m * m
    z = (a - m) * lax.rsqrt(v + 1e-5) * g + be
    z = jnp.maximum(z, 0.0)
    mask = lax.broadcasted_iota(jnp.int32, (NP, 1), 0) < N
    return jnp.where(mask, z, 0.0)


def _tc_b_body(parts_ref, degs_ref, w2_ref, g1_ref, be1_ref, y2_ref, sumz_ref):
    degs = degs_ref[...]
    nd = _norms(degs, 1)
    ns = _norms(degs, 0)
    z = _bn_relu_masked(parts_ref[...], nd, g1_ref[...], be1_ref[...])
    sumz_ref[...] = jnp.sum(z, axis=0, keepdims=True)
    y2_ref[...] = jnp.dot(z * ns[:, None], w2_ref[...],
                          preferred_element_type=jnp.float32)


def _tc_c_body(parts_ref, degs_ref, g2_ref, be2_ref, sumh_ref, sumz1_ref,
               pw0_ref, pb0_ref, pw1_ref, pb1_ref, pw2_ref, pb2_ref, out_ref):
    degs = degs_ref[...]
    nd = _norms(degs, 1)
    z2 = _bn_relu_masked(parts_ref[...], nd, g2_ref[...], be2_ref[...])
    s2 = jnp.sum(z2, axis=0, keepdims=True)
    out_ref[...] = (
        jnp.dot(sumh_ref[...], pw0_ref[...], preferred_element_type=jnp.float32)
        + pb0_ref[...]
        + jnp.dot(sumz1_ref[...], pw1_ref[...], preferred_element_type=jnp.float32)
        + pb1_ref[...]
        + jnp.dot(s2, pw2_ref[...], preferred_element_type=jnp.float32)
        + pb2_ref[...])


_tc_a = pl.pallas_call(
    _tc_a_body,
    out_shape=(jax.ShapeDtypeStruct((NP, D), jnp.float32),
               jax.ShapeDtypeStruct((1, D), jnp.float32)))

_tc_scale = pl.pallas_call(
    _tc_scale_body,
    out_shape=jax.ShapeDtypeStruct((NP, D), jnp.float32))

_tc_b = pl.pallas_call(
    _tc_b_body,
    out_shape=(jax.ShapeDtypeStruct((NP, D), jnp.float32),
               jax.ShapeDtypeStruct((1, D), jnp.float32)))

_tc_c = pl.pallas_call(
    _tc_c_body,
    out_shape=jax.ShapeDtypeStruct((1, D), jnp.float32))


def kernel(h, edge_index, W1, b1, W2, b2, g1, be1, g2, be2,
           PW0, PB0, PW1, PB1, PW2, PB2):
    pad_vals = (N + (jnp.arange(PAD, dtype=jnp.int32) % (NP - N))).astype(jnp.int32)
    srcp = jnp.concatenate([edge_index[0], pad_vals])
    dstp = jnp.concatenate([edge_index[1], pad_vals])
    src2 = srcp.reshape(ER, 128)
    dst2 = dstp.reshape(ER, 128)
    zrow = jnp.zeros((NP // NSUB, D), jnp.float32)
    zero1 = jnp.zeros((NP,), jnp.float32)
    ones128 = jnp.ones((128,), jnp.float32)

    epk = jnp.bitwise_or(jnp.left_shift(srcp, 16), dstp)

    degs = _get_deg_kernel()(src2, dst2, ones128, zero1)
    m1, sumh = _tc_a(h, W1)
    y1 = _tc_scale(m1, degs)
    parts1 = _get_agg_kernel()(y1, epk, zrow)
    y2, sumz1 = _tc_b(parts1, degs, W2, g1, be1)
    parts2 = _get_agg_kernel()(y2, epk, zrow)
    return _tc_c(parts2, degs, g2, be2, sumh, sumz1,
                 PW0, PB0, PW1, PB1, PW2, PB2)
